# Initial kernel scaffold; baseline (speedup 1.0000x reference)
import jax
import jax.numpy as jnp
from jax.experimental import pallas as pl


def kernel(x, edge_index, edge_type, lins_W, films_W, films_b, skip_W, skip_film_W, lin1_W, lin1_b):
    raise NotImplementedError("write your pallas kernel here")


# speedup vs baseline: 9.1949x; 9.1949x over previous
import functools

import jax
import jax.numpy as jnp
from jax import lax
from jax.experimental import pallas as pl
from jax.experimental.pallas import tpu as pltpu
from jax.experimental.pallas import tpu_sc as plsc

NC = 2
NS = 16
LANES = 16
NTILES = NC * NS

CHUNK = 80


def _mesh():
    return plsc.VectorSubcoreMesh(core_axis_name="c", subcore_axis_name="s")


def _make_count_kernel(N, E, R):
    per_tile = E // NTILES
    n_chunks = per_tile // CHUNK
    RN = R * N
    RNP = -(-RN // (NS * 128)) * (NS * 128)
    stripe = RNP // NS

    @functools.partial(
        pl.kernel,
        out_type=jax.ShapeDtypeStruct((NTILES, 1, stripe), jnp.float32),
        mesh=_mesh(),
        scratch_types=[
            pltpu.VMEM((CHUNK,), jnp.int32),
            pltpu.VMEM((CHUNK,), jnp.int32),
            pltpu.VMEM((CHUNK,), jnp.int32),
            pltpu.VMEM((CHUNK,), jnp.float32),
            pltpu.VMEM_SHARED((RNP,), jnp.float32),
        ],
    )
    def count_k(et_hbm, dst_hbm, ones_hbm, zeros_hbm, cnt_out,
                et_v, dst_v, idx_v, ones_v, cnt_sp):
        c = lax.axis_index("c")
        s = lax.axis_index("s")
        wid = c * NS + s

        pltpu.sync_copy(ones_hbm, ones_v)

        @pl.when(s == 0)
        def _():
            pltpu.sync_copy(zeros_hbm, cnt_sp)

        plsc.subcore_barrier()

        base = wid * per_tile

        def chunk_body(i, carry):
            off = base + i * CHUNK
            pltpu.sync_copy(et_hbm.at[pl.ds(off, CHUNK)], et_v)
            pltpu.sync_copy(dst_hbm.at[pl.ds(off, CHUNK)], dst_v)
            for t in range(CHUNK // LANES):
                sl = pl.ds(t * LANES, LANES)
                idx_v[sl] = et_v[sl] * N + dst_v[sl]
            pltpu.sync_copy(ones_v, cnt_sp.at[idx_v], add=True)
            return carry

        lax.fori_loop(0, n_chunks, chunk_body, 0)
        plsc.subcore_barrier()
        pltpu.sync_copy(cnt_sp.at[pl.ds(s * stripe, stripe)], cnt_out.at[wid, 0])

    return count_k


def _tc1_body(R, D, x_ref, w_ref, fb_ref, cnt_ref, h_ref, gb_ref, skip_ref):
    y = jnp.dot(x_ref[...], w_ref[...], preferred_element_type=jnp.float32)
    cnt = cnt_ref[...]
    for r in range(R):
        inv_r = 1.0 / jnp.maximum(cnt[:, r] + cnt[:, R + r], 1.0)
        h_ref[r] = y[:, r * D:(r + 1) * D]
        gam = (y[:, R * D + r * D: R * D + (r + 1) * D]
               + fb_ref[r, D:][None, :]) * inv_r[:, None]
        bet = (y[:, 2 * R * D + r * D: 2 * R * D + (r + 1) * D]
               + fb_ref[r, :D][None, :]) * inv_r[:, None]
        gb_ref[r, :, :D] = gam
        gb_ref[r, :, D:] = bet
    base = 3 * R * D
    xs = y[:, base:base + D]
    bet_s = y[:, base + D:base + 2 * D]
    gam_s = y[:, base + 2 * D:base + 3 * D]
    skip_ref[...] = jnp.maximum(gam_s * xs + bet_s, 0.0)


def _run_tc1(x, Wcat, films_b, cnt2, N, D, R, NB):
    grid = (N // NB,)
    K = Wcat.shape[1]
    return pl.pallas_call(
        functools.partial(_tc1_body, R, D),
        grid=grid,
        in_specs=[
            pl.BlockSpec((NB, D), lambda i: (i, 0)),
            pl.BlockSpec((D, K), lambda i: (0, 0)),
            pl.BlockSpec((R, 2 * D), lambda i: (0, 0)),
            pl.BlockSpec((NB, NC * R), lambda i: (i, 0)),
        ],
        out_specs=[
            pl.BlockSpec((R, NB, D), lambda i: (0, i, 0)),
            pl.BlockSpec((R, NB, 2 * D), lambda i: (0, i, 0)),
            pl.BlockSpec((NB, D), lambda i: (i, 0)),
        ],
        out_shape=[
            jax.ShapeDtypeStruct((R, N, D), jnp.float32),
            jax.ShapeDtypeStruct((R, N, 2 * D), jnp.float32),
            jax.ShapeDtypeStruct((N, D), jnp.float32),
        ],
    )(x, Wcat, films_b, cnt2)


def _make_edge_kernel(N, E, D, R):
    per_tile = E // NTILES
    n_chunks = per_tile // CHUNK
    rows_per_tile = N // NS
    n_zcopy = rows_per_tile // CHUNK
    z_rem = rows_per_tile - n_zcopy * CHUNK

    @functools.partial(
        pl.kernel,
        out_type=jax.ShapeDtypeStruct((NC, NS, N // NS, D), jnp.float32),
        mesh=_mesh(),
        scratch_types=[
            pltpu.VMEM((CHUNK,), jnp.int32),
            pltpu.VMEM((CHUNK,), jnp.int32),
            pltpu.VMEM((CHUNK,), jnp.int32),
            pltpu.VMEM((CHUNK,), jnp.int32),
            pltpu.VMEM((CHUNK,), jnp.int32),
            pltpu.VMEM((CHUNK, D), jnp.float32),
            pltpu.VMEM((CHUNK, 2 * D), jnp.float32),
            pltpu.VMEM((CHUNK, D), jnp.float32),
            pltpu.VMEM_SHARED((N, D), jnp.float32),
            pltpu.SemaphoreType.DMA,
            pltpu.SemaphoreType.DMA,
        ],
    )
    def edge_k(h_hbm, gb_hbm, et_hbm, src_hbm, dst_hbm, acc_out,
               et_v, src_v, dst_v, isrc_v, idst_v, hbuf, gbuf, msgbuf,
               acc_sp, sem1, sem2):
        c = lax.axis_index("c")
        s = lax.axis_index("s")
        wid = c * NS + s

        def zrow(i, carry):
            for t in range(D // LANES):
                msgbuf[i, pl.ds(t * LANES, LANES)] = jnp.zeros((LANES,), jnp.float32)
            return carry

        lax.fori_loop(0, CHUNK, zrow, 0)
        for p in range(n_zcopy):
            pltpu.sync_copy(msgbuf,
                            acc_sp.at[pl.ds(s * rows_per_tile + p * CHUNK, CHUNK)])
        if z_rem:
            pltpu.sync_copy(
                msgbuf.at[pl.ds(0, z_rem)],
                acc_sp.at[pl.ds(s * rows_per_tile + n_zcopy * CHUNK, z_rem)])
        plsc.subcore_barrier()

        base = wid * per_tile

        def chunk_body(i, carry):
            off = base + i * CHUNK
            pltpu.sync_copy(et_hbm.at[pl.ds(off, CHUNK)], et_v)
            pltpu.sync_copy(src_hbm.at[pl.ds(off, CHUNK)], src_v)
            pltpu.sync_copy(dst_hbm.at[pl.ds(off, CHUNK)], dst_v)
            for t in range(CHUNK // LANES):
                sl = pl.ds(t * LANES, LANES)
                isrc_v[sl] = et_v[sl] * N + src_v[sl]
                idst_v[sl] = et_v[sl] * N + dst_v[sl]
            cp1 = pltpu.async_copy(h_hbm.at[isrc_v], hbuf, sem1)
            cp2 = pltpu.async_copy(gb_hbm.at[idst_v], gbuf, sem2)
            cp1.wait()
            cp2.wait()

            def jbody(j, jcarry):
                for t in range(D // LANES):
                    sl = pl.ds(t * LANES, LANES)
                    g = gbuf[j, sl]
                    b = gbuf[j, pl.ds(D + t * LANES, LANES)]
                    hv = hbuf[j, sl]
                    msgbuf[j, sl] = jnp.maximum(g * hv + b, 0.0)
                return jcarry

            lax.fori_loop(0, CHUNK, jbody, 0)
            pltpu.sync_copy(msgbuf, acc_sp.at[dst_v], add=True)
            return carry

        lax.fori_loop(0, n_chunks, chunk_body, 0)
        plsc.subcore_barrier()
        pltpu.sync_copy(acc_sp.at[pl.ds(s * rows_per_tile, rows_per_tile)],
                        acc_out.at[c, s])

    return edge_k


def _tc2_body(skip_ref, acc_ref, w_ref, b_ref, o_ref):
    h = skip_ref[...] + acc_ref[0] + acc_ref[1]
    g = 0.5 * h * (1.0 + lax.erf(h * (2.0 ** -0.5)))
    o_ref[...] = (jnp.dot(g, w_ref[...], preferred_element_type=jnp.float32)
                  + b_ref[...])


def _run_tc2(skip, acc, lin1_W, lin1_b, N, D, NB):
    grid = (N // NB,)
    return pl.pallas_call(
        _tc2_body,
        grid=grid,
        in_specs=[
            pl.BlockSpec((NB, D), lambda i: (i, 0)),
            pl.BlockSpec((NC, NB, D), lambda i: (0, i, 0)),
            pl.BlockSpec((D, D), lambda i: (0, 0)),
            pl.BlockSpec((1, D), lambda i: (0, 0)),
        ],
        out_specs=pl.BlockSpec((NB, D), lambda i: (i, 0)),
        out_shape=jax.ShapeDtypeStruct((N, D), jnp.float32),
    )(skip, acc, lin1_W, lin1_b.reshape(1, D))


def kernel(x, edge_index, edge_type, lins_W, films_W, films_b,
           skip_W, skip_film_W, lin1_W, lin1_b):
    N, D = x.shape
    E = edge_type.shape[0]
    R = lins_W.shape[0]
    NB = 1000

    src = edge_index[0]
    dst = edge_index[1]

    Wcat = jnp.concatenate(
        [jnp.concatenate([lins_W[r] for r in range(R)], axis=1),
         jnp.concatenate([films_W[r][:, D:] for r in range(R)], axis=1),
         jnp.concatenate([films_W[r][:, :D] for r in range(R)], axis=1),
         skip_W,
         skip_film_W[:, :D],
         skip_film_W[:, D:]],
        axis=1)

    RNP = -(-(R * N) // (NS * 128)) * (NS * 128)
    ones_c = jnp.ones((CHUNK,), jnp.float32)
    zeros_rn = jnp.zeros((RNP,), jnp.float32)

    count_k = _make_count_kernel(N, E, R)
    cnt = count_k(edge_type, dst, ones_c, zeros_rn)
    cnt2 = (cnt.reshape(NC, RNP)[:, :R * N].reshape(NC, R, N)
            .transpose(2, 0, 1).reshape(N, NC * R))

    h_all, gb_all, skip_out = _run_tc1(x, Wcat, films_b, cnt2, N, D, R, NB)

    edge_k = _make_edge_kernel(N, E, D, R)
    acc = edge_k(h_all.reshape(R * N, D), gb_all.reshape(R * N, 2 * D),
                 edge_type, src, dst)

    return _run_tc2(skip_out, acc.reshape(NC, N, D), lin1_W, lin1_b, N, D, NB)


# speedup vs baseline: 10.2484x; 1.1146x over previous
import functools

import jax
import jax.numpy as jnp
from jax import lax
from jax.experimental import pallas as pl
from jax.experimental.pallas import tpu as pltpu
from jax.experimental.pallas import tpu_sc as plsc

NC = 2
NS = 16
LANES = 16
NTILES = NC * NS

CHUNK = 80


def _mesh():
    return plsc.VectorSubcoreMesh(core_axis_name="c", subcore_axis_name="s")


def _make_count_kernel(N, E, R):
    per_tile = E // NTILES
    n_chunks = per_tile // CHUNK
    RN = R * N
    RNP = -(-RN // (NS * 128)) * (NS * 128)
    stripe = RNP // NS

    @functools.partial(
        pl.kernel,
        out_type=jax.ShapeDtypeStruct((NTILES, 1, stripe), jnp.float32),
        mesh=_mesh(),
        scratch_types=[
            pltpu.VMEM((CHUNK,), jnp.int32),
            pltpu.VMEM((CHUNK,), jnp.int32),
            pltpu.VMEM((CHUNK,), jnp.int32),
            pltpu.VMEM((CHUNK,), jnp.float32),
            pltpu.VMEM_SHARED((RNP,), jnp.float32),
        ],
    )
    def count_k(et_hbm, dst_hbm, ones_hbm, zeros_hbm, cnt_out,
                et_v, dst_v, idx_v, ones_v, cnt_sp):
        c = lax.axis_index("c")
        s = lax.axis_index("s")
        wid = c * NS + s

        pltpu.sync_copy(ones_hbm, ones_v)

        @pl.when(s == 0)
        def _():
            pltpu.sync_copy(zeros_hbm, cnt_sp)

        plsc.subcore_barrier()

        base = wid * per_tile

        def chunk_body(i, carry):
            off = base + i * CHUNK
            pltpu.sync_copy(et_hbm.at[pl.ds(off, CHUNK)], et_v)
            pltpu.sync_copy(dst_hbm.at[pl.ds(off, CHUNK)], dst_v)
            for t in range(CHUNK // LANES):
                sl = pl.ds(t * LANES, LANES)
                idx_v[sl] = et_v[sl] * N + dst_v[sl]
            pltpu.sync_copy(ones_v, cnt_sp.at[idx_v], add=True)
            return carry

        lax.fori_loop(0, n_chunks, chunk_body, 0)
        plsc.subcore_barrier()
        pltpu.sync_copy(cnt_sp.at[pl.ds(s * stripe, stripe)], cnt_out.at[wid, 0])

    return count_k


def _tc1_body(R, D, x_ref, w_ref, fb_ref, cnt_ref, h_ref, gb_ref, skip_ref):
    y = jnp.dot(x_ref[...], w_ref[...], preferred_element_type=jnp.float32)
    cnt = cnt_ref[...]
    for r in range(R):
        inv_r = 1.0 / jnp.maximum(cnt[:, r] + cnt[:, R + r], 1.0)
        h_ref[r] = y[:, r * D:(r + 1) * D]
        gam = (y[:, R * D + r * D: R * D + (r + 1) * D]
               + fb_ref[r, D:][None, :]) * inv_r[:, None]
        bet = (y[:, 2 * R * D + r * D: 2 * R * D + (r + 1) * D]
               + fb_ref[r, :D][None, :]) * inv_r[:, None]
        gb_ref[r, :, :D] = gam
        gb_ref[r, :, D:] = bet
    base = 3 * R * D
    xs = y[:, base:base + D]
    bet_s = y[:, base + D:base + 2 * D]
    gam_s = y[:, base + 2 * D:base + 3 * D]
    skip_ref[...] = jnp.maximum(gam_s * xs + bet_s, 0.0)


def _run_tc1(x, Wcat, films_b, cnt2, N, D, R, NB):
    grid = (N // NB,)
    K = Wcat.shape[1]
    return pl.pallas_call(
        functools.partial(_tc1_body, R, D),
        grid=grid,
        in_specs=[
            pl.BlockSpec((NB, D), lambda i: (i, 0)),
            pl.BlockSpec((D, K), lambda i: (0, 0)),
            pl.BlockSpec((R, 2 * D), lambda i: (0, 0)),
            pl.BlockSpec((NB, NC * R), lambda i: (i, 0)),
        ],
        out_specs=[
            pl.BlockSpec((R, NB, D), lambda i: (0, i, 0)),
            pl.BlockSpec((R, NB, 2 * D), lambda i: (0, i, 0)),
            pl.BlockSpec((NB, D), lambda i: (i, 0)),
        ],
        out_shape=[
            jax.ShapeDtypeStruct((R, N, D), jnp.float32),
            jax.ShapeDtypeStruct((R, N, 2 * D), jnp.float32),
            jax.ShapeDtypeStruct((N, D), jnp.float32),
        ],
    )(x, Wcat, films_b, cnt2)


ECH = 48


def _edge_chunks(E):
    per_tile = E // NTILES
    n_chunks = -(-per_tile // ECH)
    n_chunks = -(-n_chunks // 4) * 4
    return per_tile, n_chunks


def _make_edge_kernel(N, E, D, R):
    per_tile, n_chunks = _edge_chunks(E)
    n_quads = n_chunks // 4
    rows_per_tile = N // NS
    NPAD = N + 16
    n_zcopy = rows_per_tile // ECH
    z_rem = rows_per_tile - n_zcopy * ECH

    @functools.partial(
        pl.kernel,
        out_type=jax.ShapeDtypeStruct((NC, NS, rows_per_tile, D), jnp.float32),
        mesh=_mesh(),
        scratch_types=[
            pltpu.VMEM((3 * ECH,), jnp.int32),
            pltpu.VMEM((3 * ECH,), jnp.int32),
            pltpu.VMEM((ECH,), jnp.int32),
            pltpu.VMEM((ECH,), jnp.int32),
            pltpu.VMEM((ECH,), jnp.int32),
            pltpu.VMEM((ECH,), jnp.int32),
            pltpu.VMEM((ECH,), jnp.int32),
            pltpu.VMEM((ECH,), jnp.int32),
            pltpu.VMEM((ECH,), jnp.int32),
            pltpu.VMEM((ECH,), jnp.int32),
            pltpu.VMEM((ECH, D), jnp.float32),
            pltpu.VMEM((ECH, D), jnp.float32),
            pltpu.VMEM((ECH, 2 * D), jnp.float32),
            pltpu.VMEM((ECH, 2 * D), jnp.float32),
            pltpu.VMEM((ECH, D), jnp.float32),
            pltpu.VMEM((ECH, D), jnp.float32),
            pltpu.VMEM_SHARED((NPAD, D), jnp.float32),
            pltpu.SemaphoreType.DMA,
            pltpu.SemaphoreType.DMA,
            pltpu.SemaphoreType.DMA,
            pltpu.SemaphoreType.DMA,
            pltpu.SemaphoreType.DMA,
            pltpu.SemaphoreType.DMA,
            pltpu.SemaphoreType.DMA,
            pltpu.SemaphoreType.DMA,
        ],
    )
    def edge_k(eb_hbm, h_hbm, gb_hbm, acc_out,
               eb0, eb1, isrc0, isrc1, idst0, idst1, sd0, sd1, sd2, sd3,
               hb0, hb1, gv0, gv1, ms0, ms1, acc_sp,
               hsem0, hsem1, gsem0, gsem1, ssem0, ssem1, ssem2, ssem3):
        c = lax.axis_index("c")
        s = lax.axis_index("s")
        wid = c * NS + s

        ebs = (eb0, eb1)
        isrcs = (isrc0, isrc1)
        idsts = (idst0, idst1)
        sds = (sd0, sd1, sd2, sd3)
        hbs = (hb0, hb1)
        gvs = (gv0, gv1)
        msgs = (ms0, ms1)
        hsems = (hsem0, hsem1)
        gsems = (gsem0, gsem1)
        ssems = (ssem0, ssem1, ssem2, ssem3)

        def prefetch(ci, b, q):
            off = (wid * n_chunks + ci) * (3 * ECH)
            pltpu.sync_copy(eb_hbm.at[pl.ds(off, 3 * ECH)], ebs[b])
            for t in range(ECH // LANES):
                sl = pl.ds(t * LANES, LANES)
                e = ebs[b][sl]
                sv = ebs[b][pl.ds(ECH + t * LANES, LANES)]
                dv = ebs[b][pl.ds(2 * ECH + t * LANES, LANES)]
                isrcs[b][sl] = e * N + sv
                idsts[b][sl] = e * N + dv
                sds[q][sl] = dv
            pltpu.async_copy(h_hbm.at[isrcs[b]], hbs[b], hsems[b])
            pltpu.async_copy(gb_hbm.at[idsts[b]], gvs[b], gsems[b])

        def wait_gathers(b):
            pltpu.make_async_copy(h_hbm.at[isrcs[b]], hbs[b], hsems[b]).wait()
            pltpu.make_async_copy(gb_hbm.at[idsts[b]], gvs[b], gsems[b]).wait()

        def start_scatter(b, q):
            pltpu.async_copy(msgs[b], acc_sp.at[sds[q]], ssems[q], add=True)

        def wait_scatter(b, q):
            pltpu.make_async_copy(msgs[b], acc_sp.at[sds[q]], ssems[q]).wait()

        def compute(b):
            def jbody(j, carry):
                for t in range(D // LANES):
                    sl = pl.ds(t * LANES, LANES)
                    g = gvs[b][j, sl]
                    be = gvs[b][j, pl.ds(D + t * LANES, LANES)]
                    hv = hbs[b][j, sl]
                    msgs[b][j, sl] = jnp.maximum(g * hv + be, 0.0)
                return carry

            lax.fori_loop(0, ECH, jbody, 0)

        def zrow(i, carry):
            for t in range(D // LANES):
                ms0[i, pl.ds(t * LANES, LANES)] = jnp.zeros((LANES,), jnp.float32)
            return carry

        lax.fori_loop(0, ECH, zrow, 0)
        for p in range(n_zcopy):
            pltpu.sync_copy(ms0,
                            acc_sp.at[pl.ds(s * rows_per_tile + p * ECH, ECH)])
        if z_rem:
            pltpu.sync_copy(
                ms0.at[pl.ds(0, z_rem)],
                acc_sp.at[pl.ds(s * rows_per_tile + n_zcopy * ECH, z_rem)])
        plsc.subcore_barrier()

        prefetch(0, 0, 0)
        prefetch(1, 1, 1)

        def quad(k, carry):
            ci0 = k * 4
            for j in range(4):
                b = j % 2
                ci = ci0 + j
                wait_gathers(b)
                if j < 2:
                    @pl.when(k > 0)
                    def _(b=b, j=j):
                        wait_scatter(b, (j - 2) % 4)
                else:
                    wait_scatter(b, (j - 2) % 4)
                compute(b)
                start_scatter(b, j)

                @pl.when(ci + 2 < n_chunks)
                def _(ci=ci, b=b, j=j):
                    prefetch(ci + 2, b, (j + 2) % 4)
            return carry

        lax.fori_loop(0, n_quads, quad, 0)
        wait_scatter(0, 2)
        wait_scatter(1, 3)
        plsc.subcore_barrier()
        pltpu.sync_copy(acc_sp.at[pl.ds(s * rows_per_tile, rows_per_tile)],
                        acc_out.at[c, s])

    return edge_k


def _tc2_body(skip_ref, acc_ref, w_ref, b_ref, o_ref):
    h = skip_ref[...] + acc_ref[0] + acc_ref[1]
    g = 0.5 * h * (1.0 + lax.erf(h * (2.0 ** -0.5)))
    o_ref[...] = (jnp.dot(g, w_ref[...], preferred_element_type=jnp.float32)
                  + b_ref[...])


def _run_tc2(skip, acc, lin1_W, lin1_b, N, D, NB):
    grid = (N // NB,)
    return pl.pallas_call(
        _tc2_body,
        grid=grid,
        in_specs=[
            pl.BlockSpec((NB, D), lambda i: (i, 0)),
            pl.BlockSpec((NC, NB, D), lambda i: (0, i, 0)),
            pl.BlockSpec((D, D), lambda i: (0, 0)),
            pl.BlockSpec((1, D), lambda i: (0, 0)),
        ],
        out_specs=pl.BlockSpec((NB, D), lambda i: (i, 0)),
        out_shape=jax.ShapeDtypeStruct((N, D), jnp.float32),
    )(skip, acc, lin1_W, lin1_b.reshape(1, D))


def kernel(x, edge_index, edge_type, lins_W, films_W, films_b,
           skip_W, skip_film_W, lin1_W, lin1_b):
    N, D = x.shape
    E = edge_type.shape[0]
    R = lins_W.shape[0]
    NB = 1000

    src = edge_index[0]
    dst = edge_index[1]

    Wcat = jnp.concatenate(
        [jnp.concatenate([lins_W[r] for r in range(R)], axis=1),
         jnp.concatenate([films_W[r][:, D:] for r in range(R)], axis=1),
         jnp.concatenate([films_W[r][:, :D] for r in range(R)], axis=1),
         skip_W,
         skip_film_W[:, :D],
         skip_film_W[:, D:]],
        axis=1)

    RNP = -(-(R * N) // (NS * 128)) * (NS * 128)
    ones_c = jnp.ones((CHUNK,), jnp.float32)
    zeros_rn = jnp.zeros((RNP,), jnp.float32)

    count_k = _make_count_kernel(N, E, R)
    cnt = count_k(edge_type, dst, ones_c, zeros_rn)
    cnt2 = (cnt.reshape(NC, RNP)[:, :R * N].reshape(NC, R, N)
            .transpose(2, 0, 1).reshape(N, NC * R))

    h_all, gb_all, skip_out = _run_tc1(x, Wcat, films_b, cnt2, N, D, R, NB)

    per_tile, n_chunks = _edge_chunks(E)
    pad = n_chunks * ECH - per_tile
    ets = jnp.pad(edge_type.reshape(NTILES, per_tile), ((0, 0), (0, pad)))
    srcs = jnp.pad(src.reshape(NTILES, per_tile), ((0, 0), (0, pad)))
    dsts = jnp.pad(dst.reshape(NTILES, per_tile), ((0, 0), (0, pad)),
                   constant_values=N)
    eb = jnp.stack([ets.reshape(NTILES, n_chunks, ECH),
                    srcs.reshape(NTILES, n_chunks, ECH),
                    dsts.reshape(NTILES, n_chunks, ECH)], axis=2).reshape(-1)

    edge_k = _make_edge_kernel(N, E, D, R)
    acc = edge_k(eb, h_all.reshape(R * N, D), gb_all.reshape(R * N, 2 * D))

    return _run_tc2(skip_out, acc.reshape(NC, N, D), lin1_W, lin1_b, N, D, NB)
